# baseline (speedup 1.0000x reference)
import functools

import jax
import jax.numpy as jnp
from jax import lax
from jax.experimental import pallas as pl
from jax.experimental.pallas import tpu as pltpu
from jax.experimental.pallas import tpu_sc as plsc

_N_EMB = 8192
_DIM = 32
_N_TOK = 16384
_B = 512
_KT = 4096
_COMMIT = 0.25

_NC = 2
_NS = 16
_NW = _NC * _NS
_BP = _N_TOK // _NW
_IC = 128
_NCH = _BP // _IC


_LW = 128


def _vq_tc_body(x_ref, xn_ref, emb2_ref, en_ref, idx_ref, minsum_ref):
    x = x_ref[...]
    xb = x.astype(jnp.bfloat16)
    xn = xn_ref[0, 0, :]
    lane = lax.broadcasted_iota(jnp.int32, (_B, _LW), 1).astype(jnp.float32)

    def tile(t, carry):
        run_min, run_idx = carry
        emb2_t = emb2_ref[pl.ds(t * _KT, _KT), :]
        en_t = en_ref[0, pl.ds(t * _KT, _KT)]
        dot2 = lax.dot_general(xb, emb2_t, (((1,), (1,)), ((), ())),
                               preferred_element_type=jnp.float32)
        d = (xn[:, None] + en_t[None, :]) - dot2
        rm = jnp.full((_B, _LW), jnp.inf, jnp.float32)
        rj = jnp.zeros((_B, _LW), jnp.float32)
        for j in range(_KT // _LW):
            c = d[:, j * _LW:(j + 1) * _LW]
            lt = c < rm
            rj = jnp.where(lt, jnp.float32(j), rj)
            rm = jnp.minimum(rm, c)
        m = jnp.min(rm, axis=1)
        idxf = rj * jnp.float32(_LW) + lane
        first_f = jnp.min(jnp.where(rm == m[:, None], idxf,
                                    jnp.float32(2 * _N_EMB)), axis=1)
        first = first_f.astype(jnp.int32)
        better = m < run_min
        run_idx = jnp.where(better, first + t * _KT, run_idx)
        run_min = jnp.where(better, m, run_min)
        run_min = run_min.astype(jnp.bfloat16).astype(jnp.float32)
        return run_min, run_idx

    init = (jnp.full((_B,), jnp.inf, jnp.float32), jnp.zeros((_B,), jnp.int32))
    run_min, run_idx = lax.fori_loop(0, _N_EMB // _KT, tile, init)
    idx_ref[0, 0, :] = run_idx

    @pl.when(pl.program_id(0) == 0)
    def _():
        minsum_ref[...] = jnp.zeros((1, 1), jnp.float32)

    minsum_ref[...] = minsum_ref[...] + jnp.sum(run_min)


def _vq_argmin_tc(flat_x, xn, embeddings2, en):
    grid = _N_TOK // _B
    idx3, minsum = pl.pallas_call(
        _vq_tc_body,
        grid=(grid,),
        in_specs=[
            pl.BlockSpec((_B, _DIM), lambda i: (i, 0)),
            pl.BlockSpec((1, 1, _B), lambda i: (i, 0, 0)),
            pl.BlockSpec((_N_EMB, _DIM), lambda i: (0, 0)),
            pl.BlockSpec((1, _N_EMB), lambda i: (0, 0)),
        ],
        out_specs=[
            pl.BlockSpec((1, 1, _B), lambda i: (i, 0, 0)),
            pl.BlockSpec((1, 1), lambda i: (0, 0)),
        ],
        out_shape=[
            jax.ShapeDtypeStruct((grid, 1, _B), jnp.int32),
            jax.ShapeDtypeStruct((1, 1), jnp.float32),
        ],
    )(flat_x, xn.reshape(grid, 1, _B), embeddings2, en.reshape(1, _N_EMB))
    return idx3.reshape(_N_TOK), minsum[0, 0]


def _gather_sc(embeddings, idx):
    mesh = plsc.VectorSubcoreMesh(core_axis_name="c", subcore_axis_name="s")

    @functools.partial(
        pl.kernel,
        mesh=mesh,
        out_type=jax.ShapeDtypeStruct((_N_TOK, _DIM), jnp.float32),
        scratch_types=[
            pltpu.VMEM((_NCH, _IC), jnp.int32),
            pltpu.VMEM((_BP, _DIM), jnp.float32),
            pltpu.SemaphoreType.DMA,
        ],
        compiler_params=pltpu.CompilerParams(use_tc_tiling_on_sc=False),
    )
    def k(table_hbm, idx_hbm, out_hbm, idx_v, rows_v, sem):
        wid = lax.axis_index("s") * _NC + lax.axis_index("c")
        base = wid * _BP
        pltpu.sync_copy(idx_hbm.at[wid], idx_v)
        for j in range(_NCH):
            pltpu.async_copy(
                table_hbm.at[idx_v.at[j]],
                rows_v.at[pl.ds(j * _IC, _IC)],
                sem,
            ).wait()
        pltpu.sync_copy(rows_v, out_hbm.at[pl.ds(base, _BP)])

    return k(embeddings, idx.reshape(_NW, _NCH, _IC))


def kernel(inputs, embeddings):
    input_shape = inputs.shape
    flat_x = inputs.reshape(-1, _DIM)
    xn = jnp.sum(flat_x ** 2, axis=1)
    en = jnp.sum(embeddings ** 2, axis=1)

    idx, minsum = _vq_argmin_tc(flat_x, xn, embeddings + embeddings, en)
    quantized = _gather_sc(embeddings, idx).reshape(input_shape)

    m = minsum / float(_N_TOK * _DIM)
    loss = m + _COMMIT * m
    quantized_ste = inputs + (quantized - inputs)
    return quantized_ste, loss


# speedup vs baseline: 1.0757x; 1.0757x over previous
import functools

import jax
import jax.numpy as jnp
from jax import lax
from jax.experimental import pallas as pl
from jax.experimental.pallas import tpu as pltpu
from jax.experimental.pallas import tpu_sc as plsc

_N_EMB = 8192
_DIM = 32
_N_TOK = 16384
_B = 1024
_KT = 4096
_COMMIT = 0.25

_NC = 2
_NS = 16
_NW = _NC * _NS
_BP = _N_TOK // _NW
_IC = 128
_NCH = _BP // _IC


_LW = 128


def _vq_tc_body(x_ref, xn_ref, emb2_ref, en_ref, idx_ref, minsum_ref):
    x = x_ref[...]
    xb = x.astype(jnp.bfloat16)
    xn = xn_ref[0, 0, :]
    lane = lax.broadcasted_iota(jnp.int32, (_B, _LW), 1).astype(jnp.float32)

    def tile(t, carry):
        run_min, run_idx = carry
        emb2_t = emb2_ref[pl.ds(t * _KT, _KT), :]
        en_t = en_ref[0, pl.ds(t * _KT, _KT)]
        dot2 = lax.dot_general(xb, emb2_t, (((1,), (1,)), ((), ())),
                               preferred_element_type=jnp.float32)
        d = (xn[:, None] + en_t[None, :]) - dot2
        rm = jnp.full((_B, _LW), jnp.inf, jnp.float32)
        rj = jnp.zeros((_B, _LW), jnp.float32)
        for j in range(_KT // _LW):
            c = d[:, j * _LW:(j + 1) * _LW]
            lt = c < rm
            rj = jnp.where(lt, jnp.float32(j), rj)
            rm = jnp.minimum(rm, c)
        m = jnp.min(rm, axis=1)
        idxf = rj * jnp.float32(_LW) + lane
        first_f = jnp.min(jnp.where(rm == m[:, None], idxf,
                                    jnp.float32(2 * _N_EMB)), axis=1)
        first = first_f.astype(jnp.int32)
        better = m < run_min
        run_idx = jnp.where(better, first + t * _KT, run_idx)
        run_min = jnp.where(better, m, run_min)
        run_min = run_min.astype(jnp.bfloat16).astype(jnp.float32)
        return run_min, run_idx

    init = (jnp.full((_B,), jnp.inf, jnp.float32), jnp.zeros((_B,), jnp.int32))
    run_min, run_idx = lax.fori_loop(0, _N_EMB // _KT, tile, init)
    idx_ref[0, 0, :] = run_idx

    @pl.when(pl.program_id(0) == 0)
    def _():
        minsum_ref[...] = jnp.zeros((1, 1), jnp.float32)

    minsum_ref[...] = minsum_ref[...] + jnp.sum(run_min)


def _vq_argmin_tc(flat_x, xn, embeddings2, en):
    grid = _N_TOK // _B
    idx3, minsum = pl.pallas_call(
        _vq_tc_body,
        grid=(grid,),
        in_specs=[
            pl.BlockSpec((_B, _DIM), lambda i: (i, 0)),
            pl.BlockSpec((1, 1, _B), lambda i: (i, 0, 0)),
            pl.BlockSpec((_N_EMB, _DIM), lambda i: (0, 0)),
            pl.BlockSpec((1, _N_EMB), lambda i: (0, 0)),
        ],
        out_specs=[
            pl.BlockSpec((1, 1, _B), lambda i: (i, 0, 0)),
            pl.BlockSpec((1, 1), lambda i: (0, 0)),
        ],
        out_shape=[
            jax.ShapeDtypeStruct((grid, 1, _B), jnp.int32),
            jax.ShapeDtypeStruct((1, 1), jnp.float32),
        ],
    )(flat_x, xn.reshape(grid, 1, _B), embeddings2, en.reshape(1, _N_EMB))
    return idx3.reshape(_N_TOK), minsum[0, 0]


def _gather_sc(embeddings, idx):
    mesh = plsc.VectorSubcoreMesh(core_axis_name="c", subcore_axis_name="s")

    @functools.partial(
        pl.kernel,
        mesh=mesh,
        out_type=jax.ShapeDtypeStruct((_N_TOK, _DIM), jnp.float32),
        scratch_types=[
            pltpu.VMEM((_NCH, _IC), jnp.int32),
            pltpu.VMEM((_BP, _DIM), jnp.float32),
            pltpu.SemaphoreType.DMA,
        ],
        compiler_params=pltpu.CompilerParams(use_tc_tiling_on_sc=False),
    )
    def k(table_hbm, idx_hbm, out_hbm, idx_v, rows_v, sem):
        wid = lax.axis_index("s") * _NC + lax.axis_index("c")
        base = wid * _BP
        pltpu.sync_copy(idx_hbm.at[wid], idx_v)
        for j in range(_NCH):
            pltpu.async_copy(
                table_hbm.at[idx_v.at[j]],
                rows_v.at[pl.ds(j * _IC, _IC)],
                sem,
            ).wait()
        pltpu.sync_copy(rows_v, out_hbm.at[pl.ds(base, _BP)])

    return k(embeddings, idx.reshape(_NW, _NCH, _IC))


def kernel(inputs, embeddings):
    input_shape = inputs.shape
    flat_x = inputs.reshape(-1, _DIM)
    xn = jnp.sum(flat_x ** 2, axis=1)
    en = jnp.sum(embeddings ** 2, axis=1)

    idx, minsum = _vq_argmin_tc(flat_x, xn, embeddings + embeddings, en)
    quantized = _gather_sc(embeddings, idx).reshape(input_shape)

    m = minsum / float(_N_TOK * _DIM)
    loss = m + _COMMIT * m
    quantized_ste = inputs + (quantized - inputs)
    return quantized_ste, loss


# speedup vs baseline: 1.1351x; 1.0552x over previous
import functools

import jax
import jax.numpy as jnp
from jax import lax
from jax.experimental import pallas as pl
from jax.experimental.pallas import tpu as pltpu
from jax.experimental.pallas import tpu_sc as plsc

_N_EMB = 8192
_DIM = 32
_N_TOK = 16384
_B = 2048
_KT = 4096
_COMMIT = 0.25

_NC = 2
_NS = 16
_NW = _NC * _NS
_BP = _N_TOK // _NW
_IC = 128
_NCH = _BP // _IC


_LW = 128


def _vq_tc_body(x_ref, xn_ref, emb2_ref, en_ref, idx_ref, minsum_ref):
    x = x_ref[...]
    xb = x.astype(jnp.bfloat16)
    xn = xn_ref[0, 0, :]
    lane = lax.broadcasted_iota(jnp.int32, (_B, _LW), 1).astype(jnp.float32)

    def tile(t, carry):
        run_min, run_idx = carry
        emb2_t = emb2_ref[pl.ds(t * _KT, _KT), :]
        en_t = en_ref[0, pl.ds(t * _KT, _KT)]
        dot2 = lax.dot_general(xb, emb2_t, (((1,), (1,)), ((), ())),
                               preferred_element_type=jnp.float32)
        d = (xn[:, None] + en_t[None, :]) - dot2
        rm = jnp.full((_B, _LW), jnp.inf, jnp.float32)
        rj = jnp.zeros((_B, _LW), jnp.float32)
        for j in range(_KT // _LW):
            c = d[:, j * _LW:(j + 1) * _LW]
            lt = c < rm
            rj = jnp.where(lt, jnp.float32(j), rj)
            rm = jnp.minimum(rm, c)
        m = jnp.min(rm, axis=1)
        idxf = rj * jnp.float32(_LW) + lane
        first_f = jnp.min(jnp.where(rm == m[:, None], idxf,
                                    jnp.float32(2 * _N_EMB)), axis=1)
        first = first_f.astype(jnp.int32)
        better = m < run_min
        run_idx = jnp.where(better, first + t * _KT, run_idx)
        run_min = jnp.where(better, m, run_min)
        run_min = run_min.astype(jnp.bfloat16).astype(jnp.float32)
        return run_min, run_idx

    init = (jnp.full((_B,), jnp.inf, jnp.float32), jnp.zeros((_B,), jnp.int32))
    run_min, run_idx = lax.fori_loop(0, _N_EMB // _KT, tile, init)
    idx_ref[0, 0, :] = run_idx

    @pl.when(pl.program_id(0) == 0)
    def _():
        minsum_ref[...] = jnp.zeros((1, 1), jnp.float32)

    minsum_ref[...] = minsum_ref[...] + jnp.sum(run_min)


def _vq_argmin_tc(flat_x, xn, embeddings2, en):
    grid = _N_TOK // _B
    idx3, minsum = pl.pallas_call(
        _vq_tc_body,
        grid=(grid,),
        in_specs=[
            pl.BlockSpec((_B, _DIM), lambda i: (i, 0)),
            pl.BlockSpec((1, 1, _B), lambda i: (i, 0, 0)),
            pl.BlockSpec((_N_EMB, _DIM), lambda i: (0, 0)),
            pl.BlockSpec((1, _N_EMB), lambda i: (0, 0)),
        ],
        out_specs=[
            pl.BlockSpec((1, 1, _B), lambda i: (i, 0, 0)),
            pl.BlockSpec((1, 1), lambda i: (0, 0)),
        ],
        out_shape=[
            jax.ShapeDtypeStruct((grid, 1, _B), jnp.int32),
            jax.ShapeDtypeStruct((1, 1), jnp.float32),
        ],
    )(flat_x, xn.reshape(grid, 1, _B), embeddings2, en.reshape(1, _N_EMB))
    return idx3.reshape(_N_TOK), minsum[0, 0]


def _gather_sc(embeddings, idx):
    mesh = plsc.VectorSubcoreMesh(core_axis_name="c", subcore_axis_name="s")

    @functools.partial(
        pl.kernel,
        mesh=mesh,
        out_type=jax.ShapeDtypeStruct((_N_TOK, _DIM), jnp.float32),
        scratch_types=[
            pltpu.VMEM((_NCH, _IC), jnp.int32),
            pltpu.VMEM((_BP, _DIM), jnp.float32),
            pltpu.SemaphoreType.DMA,
        ],
        compiler_params=pltpu.CompilerParams(use_tc_tiling_on_sc=False),
    )
    def k(table_hbm, idx_hbm, out_hbm, idx_v, rows_v, sem):
        wid = lax.axis_index("s") * _NC + lax.axis_index("c")
        base = wid * _BP
        pltpu.sync_copy(idx_hbm.at[wid], idx_v)
        for j in range(_NCH):
            pltpu.async_copy(
                table_hbm.at[idx_v.at[j]],
                rows_v.at[pl.ds(j * _IC, _IC)],
                sem,
            ).wait()
        pltpu.sync_copy(rows_v, out_hbm.at[pl.ds(base, _BP)])

    return k(embeddings, idx.reshape(_NW, _NCH, _IC))


def kernel(inputs, embeddings):
    input_shape = inputs.shape
    flat_x = inputs.reshape(-1, _DIM)
    xn = jnp.sum(flat_x ** 2, axis=1)
    en = jnp.sum(embeddings ** 2, axis=1)

    idx, minsum = _vq_argmin_tc(flat_x, xn, embeddings + embeddings, en)
    quantized = _gather_sc(embeddings, idx).reshape(input_shape)

    m = minsum / float(_N_TOK * _DIM)
    loss = m + _COMMIT * m
    quantized_ste = inputs + (quantized - inputs)
    return quantized_ste, loss


# speedup vs baseline: 1.1697x; 1.0304x over previous
import functools

import jax
import jax.numpy as jnp
from jax import lax
from jax.experimental import pallas as pl
from jax.experimental.pallas import tpu as pltpu
from jax.experimental.pallas import tpu_sc as plsc

_N_EMB = 8192
_DIM = 32
_N_TOK = 16384
_B = 4096
_KT = 4096
_COMMIT = 0.25

_NC = 2
_NS = 16
_NW = _NC * _NS
_BP = _N_TOK // _NW
_IC = 128
_NCH = _BP // _IC


_LW = 128


def _vq_tc_body(x_ref, xn_ref, emb2_ref, en_ref, idx_ref, minsum_ref):
    x = x_ref[...]
    xb = x.astype(jnp.bfloat16)
    xn = xn_ref[0, 0, :]
    lane = lax.broadcasted_iota(jnp.int32, (_B, _LW), 1).astype(jnp.float32)

    def tile(t, carry):
        run_min, run_idx = carry
        emb2_t = emb2_ref[pl.ds(t * _KT, _KT), :]
        en_t = en_ref[0, pl.ds(t * _KT, _KT)]
        dot2 = lax.dot_general(xb, emb2_t, (((1,), (1,)), ((), ())),
                               preferred_element_type=jnp.float32)
        d = (xn[:, None] + en_t[None, :]) - dot2
        rm = jnp.full((_B, _LW), jnp.inf, jnp.float32)
        rj = jnp.zeros((_B, _LW), jnp.float32)
        for j in range(_KT // _LW):
            c = d[:, j * _LW:(j + 1) * _LW]
            lt = c < rm
            rj = jnp.where(lt, jnp.float32(j), rj)
            rm = jnp.minimum(rm, c)
        m = jnp.min(rm, axis=1)
        idxf = rj * jnp.float32(_LW) + lane
        first_f = jnp.min(jnp.where(rm == m[:, None], idxf,
                                    jnp.float32(2 * _N_EMB)), axis=1)
        first = first_f.astype(jnp.int32)
        better = m < run_min
        run_idx = jnp.where(better, first + t * _KT, run_idx)
        run_min = jnp.where(better, m, run_min)
        run_min = run_min.astype(jnp.bfloat16).astype(jnp.float32)
        return run_min, run_idx

    init = (jnp.full((_B,), jnp.inf, jnp.float32), jnp.zeros((_B,), jnp.int32))
    run_min, run_idx = lax.fori_loop(0, _N_EMB // _KT, tile, init)
    idx_ref[0, 0, :] = run_idx

    @pl.when(pl.program_id(0) == 0)
    def _():
        minsum_ref[...] = jnp.zeros((1, 1), jnp.float32)

    minsum_ref[...] = minsum_ref[...] + jnp.sum(run_min)


def _vq_argmin_tc(flat_x, xn, embeddings2, en):
    grid = _N_TOK // _B
    idx3, minsum = pl.pallas_call(
        _vq_tc_body,
        grid=(grid,),
        in_specs=[
            pl.BlockSpec((_B, _DIM), lambda i: (i, 0)),
            pl.BlockSpec((1, 1, _B), lambda i: (i, 0, 0)),
            pl.BlockSpec((_N_EMB, _DIM), lambda i: (0, 0)),
            pl.BlockSpec((1, _N_EMB), lambda i: (0, 0)),
        ],
        out_specs=[
            pl.BlockSpec((1, 1, _B), lambda i: (i, 0, 0)),
            pl.BlockSpec((1, 1), lambda i: (0, 0)),
        ],
        out_shape=[
            jax.ShapeDtypeStruct((grid, 1, _B), jnp.int32),
            jax.ShapeDtypeStruct((1, 1), jnp.float32),
        ],
    )(flat_x, xn.reshape(grid, 1, _B), embeddings2, en.reshape(1, _N_EMB))
    return idx3.reshape(_N_TOK), minsum[0, 0]


def _gather_sc(embeddings, idx):
    mesh = plsc.VectorSubcoreMesh(core_axis_name="c", subcore_axis_name="s")

    @functools.partial(
        pl.kernel,
        mesh=mesh,
        out_type=jax.ShapeDtypeStruct((_N_TOK, _DIM), jnp.float32),
        scratch_types=[
            pltpu.VMEM((_NCH, _IC), jnp.int32),
            pltpu.VMEM((_BP, _DIM), jnp.float32),
            pltpu.SemaphoreType.DMA,
        ],
        compiler_params=pltpu.CompilerParams(use_tc_tiling_on_sc=False),
    )
    def k(table_hbm, idx_hbm, out_hbm, idx_v, rows_v, sem):
        wid = lax.axis_index("s") * _NC + lax.axis_index("c")
        base = wid * _BP
        pltpu.sync_copy(idx_hbm.at[wid], idx_v)
        for j in range(_NCH):
            pltpu.async_copy(
                table_hbm.at[idx_v.at[j]],
                rows_v.at[pl.ds(j * _IC, _IC)],
                sem,
            ).wait()
        pltpu.sync_copy(rows_v, out_hbm.at[pl.ds(base, _BP)])

    return k(embeddings, idx.reshape(_NW, _NCH, _IC))


def kernel(inputs, embeddings):
    input_shape = inputs.shape
    flat_x = inputs.reshape(-1, _DIM)
    xn = jnp.sum(flat_x ** 2, axis=1)
    en = jnp.sum(embeddings ** 2, axis=1)

    idx, minsum = _vq_argmin_tc(flat_x, xn, embeddings + embeddings, en)
    quantized = _gather_sc(embeddings, idx).reshape(input_shape)

    m = minsum / float(_N_TOK * _DIM)
    loss = m + _COMMIT * m
    quantized_ste = inputs + (quantized - inputs)
    return quantized_ste, loss
